# baseline (speedup 1.0000x reference)
import functools

import jax
import jax.numpy as jnp
from jax import lax
from jax.experimental import pallas as pl
from jax.experimental.pallas import tpu as pltpu
from jax.experimental.pallas import tpu_sc as plsc

F = 26
B = 4096
L = 20
V = 100000
D = 4

NC = 2
NS = 16
NW = NC * NS

ROWS = F * V
PROJ_CHUNK_ROWS = 8000
PROJ_NCHUNKS = ROWS // PROJ_CHUNK_ROWS
PROJ_CHUNKS_PER_TILE = -(-PROJ_NCHUNKS // NW)
PROJ_GROUPS = PROJ_CHUNK_ROWS // 16

BAGS = F * B
BAGS_PER_TILE = BAGS // NW
CHUNK_BAGS = 832
NCHUNKS = BAGS_PER_TILE // CHUNK_BAGS
CHUNK_IDX = CHUNK_BAGS * L


def _proj_body(tq_hbm, p_hbm, tbuf_v, p_v, sem):
    wid = lax.axis_index("s") * NC + lax.axis_index("c")
    iota4 = lax.iota(jnp.int32, 16) * D

    def chunk_body(k, carry):
        c = wid + k * NW

        @pl.when(c < PROJ_NCHUNKS)
        def _():
            row0 = c * PROJ_CHUNK_ROWS
            pltpu.sync_copy(
                tq_hbm.at[pl.ds(row0 * D, PROJ_CHUNK_ROWS * D)], tbuf_v
            )

            def group(j, cr):
                ev = iota4 + j * (16 * D)
                acc = plsc.load_gather(tbuf_v, [ev])
                for d in range(1, D):
                    acc = acc + plsc.load_gather(tbuf_v, [ev + d])
                p_v[pl.ds(j * 16, 16)] = acc
                return cr

            lax.fori_loop(0, PROJ_GROUPS, group, 0)
            pltpu.sync_copy(p_v, p_hbm.at[pl.ds(row0, PROJ_CHUNK_ROWS)])

        return carry

    lax.fori_loop(0, PROJ_CHUNKS_PER_TILE, chunk_body, 0)


def _lookup_body(idx_hbm, p_hbm, wb_hbm, out_hbm, cidx_v, vals_v, wb_v, out_v, sem):
    wid = lax.axis_index("s") * NC + lax.axis_index("c")

    pltpu.sync_copy(wb_hbm, wb_v)
    bias = wb_v[0]
    iota16 = lax.iota(jnp.int32, 16)

    def chunk_body(c, carry):
        gbag0 = wid * BAGS_PER_TILE + c * CHUNK_BAGS
        goff = gbag0 * L

        pltpu.sync_copy(idx_hbm.at[pl.ds(goff, CHUNK_IDX)], cidx_v)
        pltpu.async_copy(p_hbm.at[cidx_v], vals_v, sem).wait()

        def pool(g, cr):
            pv = (iota16 + g * 16) * L
            acc = plsc.load_gather(vals_v, [pv])
            for l in range(1, L):
                acc = acc + plsc.load_gather(vals_v, [pv + l])
            out_v[pl.ds(g * 16, 16)] = acc + bias
            return cr

        lax.fori_loop(0, CHUNK_BAGS // 16, pool, 0)

        pltpu.sync_copy(out_v, out_hbm.at[pl.ds(gbag0, CHUNK_BAGS)])
        return carry

    lax.fori_loop(0, NCHUNKS, chunk_body, 0)


_MESH = plsc.VectorSubcoreMesh(
    core_axis_name="c", subcore_axis_name="s", num_cores=NC, num_subcores=NS
)

_proj_call = functools.partial(
    pl.kernel,
    out_type=jax.ShapeDtypeStruct((ROWS,), jnp.float32),
    mesh=_MESH,
    compiler_params=pltpu.CompilerParams(needs_layout_passes=False),
    scratch_types=[
        pltpu.VMEM((PROJ_CHUNK_ROWS * D,), jnp.float32),
        pltpu.VMEM((PROJ_CHUNK_ROWS,), jnp.float32),
        pltpu.SemaphoreType.DMA,
    ],
)(_proj_body)

_lookup_call = functools.partial(
    pl.kernel,
    out_type=jax.ShapeDtypeStruct((BAGS,), jnp.float32),
    mesh=_MESH,
    compiler_params=pltpu.CompilerParams(needs_layout_passes=False),
    scratch_types=[
        pltpu.VMEM((CHUNK_IDX,), jnp.int32),
        pltpu.VMEM((CHUNK_IDX,), jnp.float32),
        pltpu.VMEM((1, 16), jnp.float32),
        pltpu.VMEM((CHUNK_BAGS,), jnp.float32),
        pltpu.SemaphoreType.DMA,
    ],
)(_lookup_body)


@jax.jit
def kernel(indices, tables, W, b):
    tq = (tables * W.reshape(1, 1, D)).reshape(ROWS * D)
    offs = (jnp.arange(F, dtype=jnp.int32) * V)[:, None, None]
    gidx = (indices + offs).reshape(F * B * L)
    bvec = jnp.broadcast_to(b.reshape(1, 1), (1, 16))
    p = _proj_call(tq)
    out = _lookup_call(gidx, p, bvec)
    return out.reshape(BAGS, 1)


# speedup vs baseline: 1.9612x; 1.9612x over previous
import functools

import jax
import jax.numpy as jnp
from jax import lax
from jax.experimental import pallas as pl
from jax.experimental.pallas import tpu as pltpu
from jax.experimental.pallas import tpu_sc as plsc

F = 26
B = 4096
L = 20
V = 100000
D = 4

NC = 2
NS = 16
NW = NC * NS

PV = 102400
BVA = 10240
ROWS = F * PV


BAGS = F * B
BAGS_PER_TILE = BAGS // NW
CHUNK_BAGS = 832
NCHUNKS = BAGS_PER_TILE // CHUNK_BAGS
CHUNK_IDX = CHUNK_BAGS * L


def _proj_tc_body(w_ref, t_ref, p_ref):
    i = pl.program_id(0)
    j = pl.program_id(1)
    acc = lax.dot_general(
        t_ref[0], w_ref[...],
        (((1,), (1,)), ((), ())),
        preferred_element_type=jnp.float32,
    )[:, 0]
    p_ref[pl.ds(i * PV + j * BVA, BVA)] = acc


def _project(tables, W):
    return pl.pallas_call(
        _proj_tc_body,
        grid=(F, PV // BVA),
        in_specs=[
            pl.BlockSpec((1, D), lambda i, j: (0, 0)),
            pl.BlockSpec((1, BVA, D), lambda i, j: (i, j, 0)),
        ],
        out_specs=pl.BlockSpec((ROWS,), lambda i, j: (0,)),
        out_shape=jax.ShapeDtypeStruct((ROWS,), jnp.float32),
    )(W, tables)


def _flatten_idx(indices):
    offs = (jnp.arange(F, dtype=jnp.int32) * PV)[:, None, None]
    return (indices + offs).reshape(F * B * L)


def _lookup_body(idx_hbm, p_hbm, wb_hbm, out_hbm, cidx_v, vals_v, wb_v, out_v, sem):
    wid = lax.axis_index("s") * NC + lax.axis_index("c")

    pltpu.sync_copy(wb_hbm, wb_v)
    bias = wb_v[0]
    iota16 = lax.iota(jnp.int32, 16)

    def chunk_body(c, carry):
        gbag0 = wid * BAGS_PER_TILE + c * CHUNK_BAGS
        goff = gbag0 * L

        pltpu.sync_copy(idx_hbm.at[pl.ds(goff, CHUNK_IDX)], cidx_v)
        pltpu.async_copy(p_hbm.at[cidx_v], vals_v, sem).wait()

        def pool(g, cr):
            pv = (iota16 + g * 16) * L
            acc = plsc.load_gather(vals_v, [pv])
            for l in range(1, L):
                acc = acc + plsc.load_gather(vals_v, [pv + l])
            out_v[pl.ds(g * 16, 16)] = acc + bias
            return cr

        lax.fori_loop(0, CHUNK_BAGS // 16, pool, 0)

        pltpu.sync_copy(out_v, out_hbm.at[pl.ds(gbag0, CHUNK_BAGS)])
        return carry

    lax.fori_loop(0, NCHUNKS, chunk_body, 0)


_MESH = plsc.VectorSubcoreMesh(
    core_axis_name="c", subcore_axis_name="s", num_cores=NC, num_subcores=NS
)

_lookup_call = functools.partial(
    pl.kernel,
    out_type=jax.ShapeDtypeStruct((BAGS,), jnp.float32),
    mesh=_MESH,
    compiler_params=pltpu.CompilerParams(needs_layout_passes=False),
    scratch_types=[
        pltpu.VMEM((CHUNK_IDX,), jnp.int32),
        pltpu.VMEM((CHUNK_IDX,), jnp.float32),
        pltpu.VMEM((1, 16), jnp.float32),
        pltpu.VMEM((CHUNK_BAGS,), jnp.float32),
        pltpu.SemaphoreType.DMA,
    ],
)(_lookup_body)


@jax.jit
def kernel(indices, tables, W, b):
    p = _project(tables, W)
    gidx = _flatten_idx(indices)
    bvec = jnp.broadcast_to(b.reshape(1, 1), (1, 16))
    out = _lookup_call(gidx, p, bvec)
    return out.reshape(BAGS, 1)
